# baseline (speedup 1.0000x reference)
import functools

import jax
import jax.numpy as jnp
from jax import lax
from jax.experimental import pallas as pl
from jax.experimental.pallas import tpu as pltpu
from jax.experimental.pallas import tpu_sc as plsc

BATCH = 4096
SEQ = 200
VOCAB = 100000
EMBED = 128
CLS = 20
CP = 32
NC, NS = 2, 16
NW = NC * NS
ROWS_PER_W = BATCH // NW
CHUNK_ROWS = 4
CHUNK_TOK = CHUNK_ROWS * SEQ
NCHUNKS = ROWS_PER_W // CHUNK_ROWS
W_TOK = ROWS_PER_W * SEQ
FOLD_R = 1000
CNT_R = 512


def _fold_body(e0, e1, e2, e3, w_ref, o_ref):
    def mm(e_ref):
        return lax.dot_general(
            e_ref[...], w_ref[...], (((1,), (1,)), ((), ())),
            preferred_element_type=jnp.float32)

    o_ref[...] = jnp.concatenate([mm(e0), mm(e1), mm(e2), mm(e3)], axis=1)


def _fold(embed, wp):
    q = VOCAB // 4 // FOLD_R
    return pl.pallas_call(
        _fold_body,
        grid=(q,),
        in_specs=[
            pl.BlockSpec((FOLD_R, EMBED), lambda i, j=j: (j * q + i, 0))
            for j in range(4)
        ] + [pl.BlockSpec((CP, EMBED), lambda i: (0, 0))],
        out_specs=pl.BlockSpec((FOLD_R, 4 * CP), lambda i: (i, 0)),
        out_shape=jax.ShapeDtypeStruct((VOCAB // 4, 4 * CP), jnp.float32),
    )(embed, embed, embed, embed, wp)


def _sc_pool(x_flat, folded, bp):
    mesh = plsc.VectorSubcoreMesh(
        core_axis_name="c", subcore_axis_name="s",
        num_cores=NC, num_subcores=NS)

    @functools.partial(
        pl.kernel,
        out_type=jax.ShapeDtypeStruct((BATCH, CP), jnp.float32),
        mesh=mesh,
        compiler_params=pltpu.CompilerParams(
            needs_layout_passes=False, use_tc_tiling_on_sc=False),
        scratch_types=[
            pltpu.VMEM((W_TOK,), jnp.int32),
            pltpu.VMEM((CHUNK_TOK, CP), jnp.float32),
            pltpu.VMEM((CHUNK_TOK, CP), jnp.float32),
            pltpu.VMEM((ROWS_PER_W, CP), jnp.float32),
            pltpu.VMEM((CP,), jnp.float32),
            pltpu.VMEM((16,), jnp.float32),
            pltpu.SemaphoreType.DMA,
            pltpu.SemaphoreType.DMA,
        ],
    )
    def body(x_hbm, folded_hbm, b_hbm, out_hbm, idx_v, rows_v0,
             rows_v1, out_v, b_v, recip_v, sem0, sem1):
        wid = lax.axis_index("s") * NC + lax.axis_index("c")
        pltpu.sync_copy(b_hbm, b_v)
        pltpu.sync_copy(x_hbm.at[pl.ds(wid * W_TOK, W_TOK)], idx_v)
        b0 = b_v[pl.ds(0, 16)]
        b1 = b_v[pl.ds(16, 16)]
        zeros16i = jnp.zeros((16,), jnp.int32)
        lanes = lax.iota(jnp.int32, 16)
        lane_base = (lanes % CHUNK_ROWS) * SEQ

        def stage_body(i, carry):
            for u in range(2):
                v = idx_v[pl.ds(i * 32 + u * 16, 16)]
                p = (v % (VOCAB // 4)) * 4 + v // (VOCAB // 4)
                idx_v[pl.ds(i * 32 + u * 16, 16)] = p
            return carry

        lax.fori_loop(0, W_TOK // 32, stage_body, 0)

        def start_gather(c, rows_v, sem):
            pltpu.make_async_copy(
                folded_hbm.at[idx_v.at[pl.ds(c * CHUNK_TOK, CHUNK_TOK)]],
                rows_v, sem).start()

        def process(c, rows_v, sem):
            pltpu.make_async_copy(
                folded_hbm.at[idx_v.at[pl.ds(c * CHUNK_TOK, CHUNK_TOK)]],
                rows_v, sem).wait()
            cbase = c * CHUNK_TOK

            def tok_body(t, carry):
                cnt = carry[0]
                v = plsc.load_gather(idx_v, [lane_base + (cbase + t)])
                cnt = cnt + jnp.where(v != 0, 1, 0)
                accs = [cnt]
                for r in range(CHUNK_ROWS):
                    accs.append(carry[1 + 2 * r] + rows_v[r * SEQ + t, 0:16])
                    accs.append(carry[2 + 2 * r] + rows_v[r * SEQ + t, 16:32])
                return tuple(accs)

            init = (jnp.zeros((16,), jnp.int32),) + tuple(
                jnp.zeros((16,), jnp.float32) for _ in range(2 * CHUNK_ROWS))
            res = lax.fori_loop(0, SEQ, tok_body, init)
            recip_v[...] = 1.0 / res[0].astype(jnp.float32)
            for r in range(CHUNK_ROWS):
                row = c * CHUNK_ROWS + r
                rsp = plsc.load_gather(recip_v, [zeros16i + r])
                out_v[row, 0:16] = res[1 + 2 * r] * rsp + b0
                out_v[row, 16:32] = res[2 + 2 * r] * rsp + b1

        start_gather(0, rows_v0, sem0)

        def super_body(g, carry):
            start_gather(2 * g + 1, rows_v1, sem1)
            process(2 * g, rows_v0, sem0)

            @pl.when(g < NCHUNKS // 2 - 1)
            def _():
                start_gather(2 * g + 2, rows_v0, sem0)

            process(2 * g + 1, rows_v1, sem1)
            return carry

        lax.fori_loop(0, NCHUNKS // 2, super_body, 0)
        pltpu.sync_copy(out_v, out_hbm.at[pl.ds(wid * ROWS_PER_W,
                                                ROWS_PER_W)])

    return body(x_flat, folded, bp)


def kernel(X, embed_weight, W, b):
    x_flat = X.astype(jnp.int32).reshape(BATCH * SEQ)
    wp = jnp.zeros((CP, EMBED), jnp.float32).at[:CLS].set(W)
    bp = jnp.zeros((CP,), jnp.float32).at[:CLS].set(b)
    folded = _fold(embed_weight, wp).reshape(VOCAB, CP)
    out = _sc_pool(x_flat, folded, bp)
    return out[:, :CLS]


# speedup vs baseline: 1.4812x; 1.4812x over previous
import functools

import jax
import jax.numpy as jnp
from jax import lax
from jax.experimental import pallas as pl
from jax.experimental.pallas import tpu as pltpu
from jax.experimental.pallas import tpu_sc as plsc

BATCH = 4096
SEQ = 200
VOCAB = 100000
EMBED = 128
CLS = 20
CP = 32
NC, NS = 2, 16
NW = NC * NS
ROWS_PER_W = BATCH // NW
CHUNK_ROWS = 4
CHUNK_TOK = CHUNK_ROWS * SEQ
NCHUNKS = ROWS_PER_W // CHUNK_ROWS
W_TOK = ROWS_PER_W * SEQ
FOLD_R = 1000
CNT_R = 512


def _fold_body(e0, e1, e2, e3, w_ref, o_ref):
    def mm(e_ref):
        return lax.dot_general(
            e_ref[...], w_ref[...], (((1,), (1,)), ((), ())),
            preferred_element_type=jnp.float32)

    o_ref[...] = jnp.concatenate([mm(e0), mm(e1), mm(e2), mm(e3)], axis=1)


def _fold(embed, wp):
    q = VOCAB // 4 // FOLD_R
    return pl.pallas_call(
        _fold_body,
        grid=(q,),
        in_specs=[
            pl.BlockSpec((FOLD_R, EMBED), lambda i, j=j: (j * q + i, 0))
            for j in range(4)
        ] + [pl.BlockSpec((CP, EMBED), lambda i: (0, 0))],
        out_specs=pl.BlockSpec((FOLD_R, 4 * CP), lambda i: (i, 0)),
        out_shape=jax.ShapeDtypeStruct((VOCAB // 4, 4 * CP), jnp.float32),
    )(embed, embed, embed, embed, wp)


def _sc_pool(x_flat, folded, bp):
    mesh = plsc.VectorSubcoreMesh(
        core_axis_name="c", subcore_axis_name="s",
        num_cores=NC, num_subcores=NS)

    @functools.partial(
        pl.kernel,
        out_type=jax.ShapeDtypeStruct((BATCH, CP), jnp.float32),
        mesh=mesh,
        compiler_params=pltpu.CompilerParams(
            needs_layout_passes=False, use_tc_tiling_on_sc=False),
        scratch_types=[
            pltpu.VMEM((W_TOK,), jnp.int32),
            pltpu.VMEM((CHUNK_TOK, CP), jnp.float32),
            pltpu.VMEM((CHUNK_TOK, CP), jnp.float32),
            pltpu.VMEM((ROWS_PER_W, CP), jnp.float32),
            pltpu.VMEM((CP,), jnp.float32),
            pltpu.VMEM((16,), jnp.float32),
            pltpu.SemaphoreType.DMA,
            pltpu.SemaphoreType.DMA,
        ],
    )
    def body(x_hbm, folded_hbm, b_hbm, out_hbm, idx_v, rows_v0,
             rows_v1, out_v, b_v, recip_v, sem0, sem1):
        wid = lax.axis_index("s") * NC + lax.axis_index("c")
        pltpu.sync_copy(b_hbm, b_v)
        pltpu.sync_copy(x_hbm.at[pl.ds(wid * W_TOK, W_TOK)], idx_v)
        b0 = b_v[pl.ds(0, 16)]
        b1 = b_v[pl.ds(16, 16)]
        zeros16i = jnp.zeros((16,), jnp.int32)
        lanes = lax.iota(jnp.int32, 16)
        lane_base = (lanes % CHUNK_ROWS) * SEQ

        def stage_body(i, carry):
            for u in range(2):
                v = idx_v[pl.ds(i * 32 + u * 16, 16)]
                one = jnp.int32(1)
                zero = jnp.int32(0)
                q = (jnp.where(v >= VOCAB // 4, one, zero)
                     + jnp.where(v >= VOCAB // 2, one, zero)
                     + jnp.where(v >= 3 * (VOCAB // 4), one, zero))
                p = (v - q * (VOCAB // 4)) * 4 + q
                idx_v[pl.ds(i * 32 + u * 16, 16)] = p
            return carry

        lax.fori_loop(0, W_TOK // 32, stage_body, 0)

        def start_gather(c, rows_v, sem):
            pltpu.make_async_copy(
                folded_hbm.at[idx_v.at[pl.ds(c * CHUNK_TOK, CHUNK_TOK)]],
                rows_v, sem).start()

        def process(c, rows_v, sem):
            pltpu.make_async_copy(
                folded_hbm.at[idx_v.at[pl.ds(c * CHUNK_TOK, CHUNK_TOK)]],
                rows_v, sem).wait()
            cbase = c * CHUNK_TOK

            def tok_body(t, carry):
                cnt = carry[0]
                v = plsc.load_gather(idx_v, [lane_base + (cbase + t)])
                cnt = cnt + jnp.where(v != 0, 1, 0)
                accs = [cnt]
                for r in range(CHUNK_ROWS):
                    accs.append(carry[1 + 2 * r] + rows_v[r * SEQ + t, 0:16])
                    accs.append(carry[2 + 2 * r] + rows_v[r * SEQ + t, 16:32])
                return tuple(accs)

            init = (jnp.zeros((16,), jnp.int32),) + tuple(
                jnp.zeros((16,), jnp.float32) for _ in range(2 * CHUNK_ROWS))
            res = lax.fori_loop(0, SEQ, tok_body, init)
            recip_v[...] = 1.0 / res[0].astype(jnp.float32)
            for r in range(CHUNK_ROWS):
                row = c * CHUNK_ROWS + r
                rsp = plsc.load_gather(recip_v, [zeros16i + r])
                out_v[row, 0:16] = res[1 + 2 * r] * rsp + b0
                out_v[row, 16:32] = res[2 + 2 * r] * rsp + b1

        start_gather(0, rows_v0, sem0)

        def super_body(g, carry):
            start_gather(2 * g + 1, rows_v1, sem1)
            process(2 * g, rows_v0, sem0)

            @pl.when(g < NCHUNKS // 2 - 1)
            def _():
                start_gather(2 * g + 2, rows_v0, sem0)

            process(2 * g + 1, rows_v1, sem1)
            return carry

        lax.fori_loop(0, NCHUNKS // 2, super_body, 0)
        pltpu.sync_copy(out_v, out_hbm.at[pl.ds(wid * ROWS_PER_W,
                                                ROWS_PER_W)])

    return body(x_flat, folded, bp)


def kernel(X, embed_weight, W, b):
    x_flat = X.astype(jnp.int32).reshape(BATCH * SEQ)
    wp = jnp.zeros((CP, EMBED), jnp.float32).at[:CLS].set(W)
    bp = jnp.zeros((CP,), jnp.float32).at[:CLS].set(b)
    folded = _fold(embed_weight, wp).reshape(VOCAB, CP)
    out = _sc_pool(x_flat, folded, bp)
    return out[:, :CLS]


# speedup vs baseline: 1.4916x; 1.0070x over previous
import functools

import jax
import jax.numpy as jnp
from jax import lax
from jax.experimental import pallas as pl
from jax.experimental.pallas import tpu as pltpu
from jax.experimental.pallas import tpu_sc as plsc

BATCH = 4096
SEQ = 200
VOCAB = 100000
EMBED = 128
CLS = 20
CP = 32
NC, NS = 2, 16
NW = NC * NS
ROWS_PER_W = BATCH // NW
CHUNK_ROWS = 4
CHUNK_TOK = CHUNK_ROWS * SEQ
NCHUNKS = ROWS_PER_W // CHUNK_ROWS
W_TOK = ROWS_PER_W * SEQ
FOLD_R = 1000
CNT_R = 512


def _fold_body(e0, e1, e2, e3, w_ref, o_ref):
    i = pl.program_id(0)
    lane = lax.broadcasted_iota(jnp.int32, (FOLD_R, CP), 1)
    row = lax.broadcasted_iota(jnp.int32, (FOLD_R, CP), 0)

    def mm(e_ref, j):
        g = lax.dot_general(
            e_ref[...], w_ref[...], (((1,), (1,)), ((), ())),
            preferred_element_type=jnp.float32)
        if j == 0:
            ind = jnp.where((lane == CLS) & ((row + i * FOLD_R) != 0),
                            1.0, 0.0)
        else:
            ind = jnp.where(lane == CLS, 1.0, 0.0)
        return g + ind

    o_ref[...] = jnp.concatenate(
        [mm(e0, 0), mm(e1, 1), mm(e2, 2), mm(e3, 3)], axis=1)


def _fold(embed, wp):
    q = VOCAB // 4 // FOLD_R
    return pl.pallas_call(
        _fold_body,
        grid=(q,),
        in_specs=[
            pl.BlockSpec((FOLD_R, EMBED), lambda i, j=j: (j * q + i, 0))
            for j in range(4)
        ] + [pl.BlockSpec((CP, EMBED), lambda i: (0, 0))],
        out_specs=pl.BlockSpec((FOLD_R, 4 * CP), lambda i: (i, 0)),
        out_shape=jax.ShapeDtypeStruct((VOCAB // 4, 4 * CP), jnp.float32),
    )(embed, embed, embed, embed, wp)


def _sc_pool(x_flat, folded, bp):
    mesh = plsc.VectorSubcoreMesh(
        core_axis_name="c", subcore_axis_name="s",
        num_cores=NC, num_subcores=NS)

    @functools.partial(
        pl.kernel,
        out_type=jax.ShapeDtypeStruct((BATCH, CP), jnp.float32),
        mesh=mesh,
        compiler_params=pltpu.CompilerParams(
            needs_layout_passes=False, use_tc_tiling_on_sc=False),
        scratch_types=[
            pltpu.VMEM((W_TOK,), jnp.int32),
            pltpu.VMEM((CHUNK_TOK, CP), jnp.float32),
            pltpu.VMEM((CHUNK_TOK, CP), jnp.float32),
            pltpu.VMEM((ROWS_PER_W, CP), jnp.float32),
            pltpu.VMEM((CP,), jnp.float32),
            pltpu.VMEM((16,), jnp.float32),
            pltpu.SemaphoreType.DMA,
            pltpu.SemaphoreType.DMA,
        ],
    )
    def body(x_hbm, folded_hbm, b_hbm, out_hbm, idx_v, rows_v0,
             rows_v1, out_v, b_v, recip_v, sem0, sem1):
        wid = lax.axis_index("s") * NC + lax.axis_index("c")
        pltpu.sync_copy(b_hbm, b_v)
        pltpu.sync_copy(x_hbm.at[pl.ds(wid * W_TOK, W_TOK)], idx_v)
        b0 = b_v[pl.ds(0, 16)]
        b1 = b_v[pl.ds(16, 16)]
        zeros16i = jnp.zeros((16,), jnp.int32)

        def stage_body(i, carry):
            for u in range(2):
                v = idx_v[pl.ds(i * 32 + u * 16, 16)]
                one = jnp.int32(1)
                zero = jnp.int32(0)
                q = (jnp.where(v >= VOCAB // 4, one, zero)
                     + jnp.where(v >= VOCAB // 2, one, zero)
                     + jnp.where(v >= 3 * (VOCAB // 4), one, zero))
                p = (v - q * (VOCAB // 4)) * 4 + q
                idx_v[pl.ds(i * 32 + u * 16, 16)] = p
            return carry

        lax.fori_loop(0, W_TOK // 32, stage_body, 0)

        def start_gather(c, rows_v, sem):
            pltpu.make_async_copy(
                folded_hbm.at[idx_v.at[pl.ds(c * CHUNK_TOK, CHUNK_TOK)]],
                rows_v, sem).start()

        def process(c, rows_v, sem):
            pltpu.make_async_copy(
                folded_hbm.at[idx_v.at[pl.ds(c * CHUNK_TOK, CHUNK_TOK)]],
                rows_v, sem).wait()

            def tok_body(t, carry):
                accs = []
                for r in range(CHUNK_ROWS):
                    accs.append(carry[2 * r] + rows_v[r * SEQ + t, 0:16])
                    accs.append(carry[2 * r + 1] + rows_v[r * SEQ + t, 16:32])
                return tuple(accs)

            init = tuple(
                jnp.zeros((16,), jnp.float32) for _ in range(2 * CHUNK_ROWS))
            res = lax.fori_loop(0, SEQ, tok_body, init)
            for r in range(CHUNK_ROWS):
                row = c * CHUNK_ROWS + r
                recip_v[...] = res[2 * r + 1]
                rsp = 1.0 / plsc.load_gather(recip_v, [zeros16i + (CLS - 16)])
                out_v[row, 0:16] = res[2 * r] * rsp + b0
                out_v[row, 16:32] = res[2 * r + 1] * rsp + b1

        start_gather(0, rows_v0, sem0)

        def super_body(g, carry):
            start_gather(2 * g + 1, rows_v1, sem1)
            process(2 * g, rows_v0, sem0)

            @pl.when(g < NCHUNKS // 2 - 1)
            def _():
                start_gather(2 * g + 2, rows_v0, sem0)

            process(2 * g + 1, rows_v1, sem1)
            return carry

        lax.fori_loop(0, NCHUNKS // 2, super_body, 0)
        pltpu.sync_copy(out_v, out_hbm.at[pl.ds(wid * ROWS_PER_W,
                                                ROWS_PER_W)])

    return body(x_flat, folded, bp)


def kernel(X, embed_weight, W, b):
    x_flat = X.astype(jnp.int32).reshape(BATCH * SEQ)
    wp = jnp.zeros((CP, EMBED), jnp.float32).at[:CLS].set(W)
    bp = jnp.zeros((CP,), jnp.float32).at[:CLS].set(b)
    folded = _fold(embed_weight, wp).reshape(VOCAB, CP)
    out = _sc_pool(x_flat, folded, bp)
    return out[:, :CLS]


# speedup vs baseline: 1.6260x; 1.0901x over previous
import functools

import jax
import jax.numpy as jnp
from jax import lax
from jax.experimental import pallas as pl
from jax.experimental.pallas import tpu as pltpu
from jax.experimental.pallas import tpu_sc as plsc

BATCH = 4096
SEQ = 200
VOCAB = 100000
EMBED = 128
CLS = 20
CP = 32
NC, NS = 2, 16
NW = NC * NS
ROWS_PER_W = BATCH // NW
CHUNK_ROWS = 4
CHUNK_TOK = CHUNK_ROWS * SEQ
NCHUNKS = ROWS_PER_W // CHUNK_ROWS
W_TOK = ROWS_PER_W * SEQ
FOLD_R = 5000
CNT_R = 512


def _fold_body(e0, e1, e2, e3, w_ref, o_ref):
    i = pl.program_id(0)
    lane = lax.broadcasted_iota(jnp.int32, (FOLD_R, CP), 1)
    row = lax.broadcasted_iota(jnp.int32, (FOLD_R, CP), 0)

    def mm(e_ref, j):
        g = lax.dot_general(
            e_ref[...], w_ref[...], (((1,), (1,)), ((), ())),
            preferred_element_type=jnp.float32)
        if j == 0:
            ind = jnp.where((lane == CLS) & ((row + i * FOLD_R) != 0),
                            1.0, 0.0)
        else:
            ind = jnp.where(lane == CLS, 1.0, 0.0)
        return g + ind

    o_ref[...] = jnp.concatenate(
        [mm(e0, 0), mm(e1, 1), mm(e2, 2), mm(e3, 3)], axis=1)


def _fold(embed, wp):
    q = VOCAB // 4 // FOLD_R
    return pl.pallas_call(
        _fold_body,
        grid=(q,),
        in_specs=[
            pl.BlockSpec((FOLD_R, EMBED), lambda i, j=j: (j * q + i, 0))
            for j in range(4)
        ] + [pl.BlockSpec((CP, EMBED), lambda i: (0, 0))],
        out_specs=pl.BlockSpec((FOLD_R, 4 * CP), lambda i: (i, 0)),
        out_shape=jax.ShapeDtypeStruct((VOCAB // 4, 4 * CP), jnp.float32),
    )(embed, embed, embed, embed, wp)


def _sc_pool(x_flat, folded, bp):
    mesh = plsc.VectorSubcoreMesh(
        core_axis_name="c", subcore_axis_name="s",
        num_cores=NC, num_subcores=NS)

    @functools.partial(
        pl.kernel,
        out_type=jax.ShapeDtypeStruct((BATCH, CP), jnp.float32),
        mesh=mesh,
        compiler_params=pltpu.CompilerParams(
            needs_layout_passes=False, use_tc_tiling_on_sc=False),
        scratch_types=[
            pltpu.VMEM((W_TOK,), jnp.int32),
            pltpu.VMEM((CHUNK_TOK, CP), jnp.float32),
            pltpu.VMEM((CHUNK_TOK, CP), jnp.float32),
            pltpu.VMEM((ROWS_PER_W, CP), jnp.float32),
            pltpu.VMEM((CP,), jnp.float32),
            pltpu.VMEM((16,), jnp.float32),
            pltpu.SemaphoreType.DMA,
            pltpu.SemaphoreType.DMA,
        ],
    )
    def body(x_hbm, folded_hbm, b_hbm, out_hbm, idx_v, rows_v0,
             rows_v1, out_v, b_v, recip_v, sem0, sem1):
        wid = lax.axis_index("s") * NC + lax.axis_index("c")
        pltpu.sync_copy(b_hbm, b_v)
        pltpu.sync_copy(x_hbm.at[pl.ds(wid * W_TOK, W_TOK)], idx_v)
        b0 = b_v[pl.ds(0, 16)]
        b1 = b_v[pl.ds(16, 16)]
        zeros16i = jnp.zeros((16,), jnp.int32)

        def stage_body(i, carry):
            for u in range(2):
                v = idx_v[pl.ds(i * 32 + u * 16, 16)]
                one = jnp.int32(1)
                zero = jnp.int32(0)
                q = (jnp.where(v >= VOCAB // 4, one, zero)
                     + jnp.where(v >= VOCAB // 2, one, zero)
                     + jnp.where(v >= 3 * (VOCAB // 4), one, zero))
                p = (v - q * (VOCAB // 4)) * 4 + q
                idx_v[pl.ds(i * 32 + u * 16, 16)] = p
            return carry

        lax.fori_loop(0, W_TOK // 32, stage_body, 0)

        def start_gather(c, rows_v, sem):
            pltpu.make_async_copy(
                folded_hbm.at[idx_v.at[pl.ds(c * CHUNK_TOK, CHUNK_TOK)]],
                rows_v, sem).start()

        def process(c, rows_v, sem):
            pltpu.make_async_copy(
                folded_hbm.at[idx_v.at[pl.ds(c * CHUNK_TOK, CHUNK_TOK)]],
                rows_v, sem).wait()

            def tok_body(t, carry):
                accs = []
                for r in range(CHUNK_ROWS):
                    accs.append(carry[2 * r] + rows_v[r * SEQ + t, 0:16])
                    accs.append(carry[2 * r + 1] + rows_v[r * SEQ + t, 16:32])
                return tuple(accs)

            init = tuple(
                jnp.zeros((16,), jnp.float32) for _ in range(2 * CHUNK_ROWS))
            res = lax.fori_loop(0, SEQ, tok_body, init)
            for r in range(CHUNK_ROWS):
                row = c * CHUNK_ROWS + r
                recip_v[...] = res[2 * r + 1]
                rsp = 1.0 / plsc.load_gather(recip_v, [zeros16i + (CLS - 16)])
                out_v[row, 0:16] = res[2 * r] * rsp + b0
                out_v[row, 16:32] = res[2 * r + 1] * rsp + b1

        start_gather(0, rows_v0, sem0)

        def super_body(g, carry):
            start_gather(2 * g + 1, rows_v1, sem1)
            process(2 * g, rows_v0, sem0)

            @pl.when(g < NCHUNKS // 2 - 1)
            def _():
                start_gather(2 * g + 2, rows_v0, sem0)

            process(2 * g + 1, rows_v1, sem1)
            return carry

        lax.fori_loop(0, NCHUNKS // 2, super_body, 0)
        pltpu.sync_copy(out_v, out_hbm.at[pl.ds(wid * ROWS_PER_W,
                                                ROWS_PER_W)])

    return body(x_flat, folded, bp)


def kernel(X, embed_weight, W, b):
    x_flat = X.astype(jnp.int32).reshape(BATCH * SEQ)
    wp = jnp.zeros((CP, EMBED), jnp.float32).at[:CLS].set(W)
    bp = jnp.zeros((CP,), jnp.float32).at[:CLS].set(b)
    folded = _fold(embed_weight, wp).reshape(VOCAB, CP)
    out = _sc_pool(x_flat, folded, bp)
    return out[:, :CLS]


# speedup vs baseline: 1.7238x; 1.0602x over previous
import functools

import jax
import jax.numpy as jnp
from jax import lax
from jax.experimental import pallas as pl
from jax.experimental.pallas import tpu as pltpu
from jax.experimental.pallas import tpu_sc as plsc

BATCH = 4096
SEQ = 200
VOCAB = 100000
EMBED = 128
CLS = 20
CP = 32
NC, NS = 2, 16
NW = NC * NS
ROWS_PER_W = BATCH // NW
CHUNK_ROWS = 4
CHUNK_TOK = CHUNK_ROWS * SEQ
NCHUNKS = ROWS_PER_W // CHUNK_ROWS
W_TOK = ROWS_PER_W * SEQ
FOLD_R = 5000
CNT_R = 512


def _fold_body(e0, e1, e2, e3, w_ref, o_ref):
    i = pl.program_id(0)
    lane = lax.broadcasted_iota(jnp.int32, (FOLD_R, CP), 1)
    row = lax.broadcasted_iota(jnp.int32, (FOLD_R, CP), 0)

    def mm(e_ref, j):
        g = lax.dot_general(
            e_ref[...], w_ref[...], (((1,), (1,)), ((), ())),
            preferred_element_type=jnp.float32)
        if j == 0:
            ind = jnp.where((lane == CLS) & ((row + i * FOLD_R) != 0),
                            1.0, 0.0)
        else:
            ind = jnp.where(lane == CLS, 1.0, 0.0)
        return g + ind

    o_ref[...] = jnp.concatenate(
        [mm(e0, 0), mm(e1, 1), mm(e2, 2), mm(e3, 3)], axis=1)


def _fold(embed, wp):
    q = VOCAB // 4 // FOLD_R
    return pl.pallas_call(
        _fold_body,
        grid=(q,),
        in_specs=[
            pl.BlockSpec((FOLD_R, EMBED), lambda i, j=j: (j * q + i, 0))
            for j in range(4)
        ] + [pl.BlockSpec((CP, EMBED), lambda i: (0, 0))],
        out_specs=pl.BlockSpec((FOLD_R, 4 * CP), lambda i: (i, 0)),
        out_shape=jax.ShapeDtypeStruct((VOCAB // 4, 4 * CP), jnp.float32),
    )(embed, embed, embed, embed, wp)


def _sc_pool(x_flat, folded, bp):
    mesh = plsc.VectorSubcoreMesh(
        core_axis_name="c", subcore_axis_name="s",
        num_cores=NC, num_subcores=NS)

    @functools.partial(
        pl.kernel,
        out_type=jax.ShapeDtypeStruct((BATCH, CP), jnp.float32),
        mesh=mesh,
        compiler_params=pltpu.CompilerParams(
            needs_layout_passes=False, use_tc_tiling_on_sc=False),
        scratch_types=[
            pltpu.VMEM((W_TOK,), jnp.int32),
            pltpu.VMEM((CHUNK_TOK, CP), jnp.float32),
            pltpu.VMEM((CHUNK_TOK, CP), jnp.float32),
            pltpu.VMEM((ROWS_PER_W, CP), jnp.float32),
            pltpu.VMEM((CP,), jnp.float32),
            pltpu.VMEM((16,), jnp.float32),
            pltpu.SemaphoreType.DMA,
            pltpu.SemaphoreType.DMA,
        ],
    )
    def body(x_hbm, folded_hbm, b_hbm, out_hbm, idx_v, rows_v0,
             rows_v1, out_v, b_v, recip_v, sem0, sem1):
        wid = lax.axis_index("s") * NC + lax.axis_index("c")
        pltpu.sync_copy(b_hbm, b_v)
        pltpu.sync_copy(x_hbm.at[pl.ds(wid * W_TOK, W_TOK)], idx_v)
        b0 = b_v[pl.ds(0, 16)]
        b1 = b_v[pl.ds(16, 16)]
        zeros16i = jnp.zeros((16,), jnp.int32)

        def stage(c):
            cb = c * CHUNK_TOK

            def stage_body(i, carry):
                for u in range(2):
                    v = idx_v[pl.ds(cb + i * 32 + u * 16, 16)]
                    one = jnp.int32(1)
                    zero = jnp.int32(0)
                    q = (jnp.where(v >= VOCAB // 4, one, zero)
                         + jnp.where(v >= VOCAB // 2, one, zero)
                         + jnp.where(v >= 3 * (VOCAB // 4), one, zero))
                    p = (v - q * (VOCAB // 4)) * 4 + q
                    idx_v[pl.ds(cb + i * 32 + u * 16, 16)] = p
                return carry

            lax.fori_loop(0, CHUNK_TOK // 32, stage_body, 0)

        def start_gather(c, rows_v, sem):
            pltpu.make_async_copy(
                folded_hbm.at[idx_v.at[pl.ds(c * CHUNK_TOK, CHUNK_TOK)]],
                rows_v, sem).start()

        def process(c, rows_v, sem):
            pltpu.make_async_copy(
                folded_hbm.at[idx_v.at[pl.ds(c * CHUNK_TOK, CHUNK_TOK)]],
                rows_v, sem).wait()

            def tok_body(t, carry):
                accs = []
                for r in range(CHUNK_ROWS):
                    accs.append(carry[2 * r] + rows_v[r * SEQ + t, 0:16])
                    accs.append(carry[2 * r + 1] + rows_v[r * SEQ + t, 16:32])
                return tuple(accs)

            init = tuple(
                jnp.zeros((16,), jnp.float32) for _ in range(2 * CHUNK_ROWS))
            res = lax.fori_loop(0, SEQ, tok_body, init)
            for r in range(CHUNK_ROWS):
                row = c * CHUNK_ROWS + r
                recip_v[...] = res[2 * r + 1]
                rsp = 1.0 / plsc.load_gather(recip_v, [zeros16i + (CLS - 16)])
                out_v[row, 0:16] = res[2 * r] * rsp + b0
                out_v[row, 16:32] = res[2 * r + 1] * rsp + b1

        stage(0)
        start_gather(0, rows_v0, sem0)
        stage(1)

        def super_body(g, carry):
            start_gather(2 * g + 1, rows_v1, sem1)

            @pl.when(g < NCHUNKS // 2 - 1)
            def _():
                stage(2 * g + 2)

            process(2 * g, rows_v0, sem0)

            @pl.when(g < NCHUNKS // 2 - 1)
            def _():
                start_gather(2 * g + 2, rows_v0, sem0)
                stage(2 * g + 3)

            process(2 * g + 1, rows_v1, sem1)
            return carry

        lax.fori_loop(0, NCHUNKS // 2, super_body, 0)
        pltpu.sync_copy(out_v, out_hbm.at[pl.ds(wid * ROWS_PER_W,
                                                ROWS_PER_W)])

    return body(x_flat, folded, bp)


def kernel(X, embed_weight, W, b):
    x_flat = X.astype(jnp.int32).reshape(BATCH * SEQ)
    wp = jnp.zeros((CP, EMBED), jnp.float32).at[:CLS].set(W)
    bp = jnp.zeros((CP,), jnp.float32).at[:CLS].set(b)
    folded = _fold(embed_weight, wp).reshape(VOCAB, CP)
    out = _sc_pool(x_flat, folded, bp)
    return out[:, :CLS]


# speedup vs baseline: 1.7412x; 1.0101x over previous
import functools

import jax
import jax.numpy as jnp
from jax import lax
from jax.experimental import pallas as pl
from jax.experimental.pallas import tpu as pltpu
from jax.experimental.pallas import tpu_sc as plsc

BATCH = 4096
SEQ = 200
VOCAB = 100000
EMBED = 128
CLS = 20
CP = 32
NC, NS = 2, 16
NW = NC * NS
ROWS_PER_W = BATCH // NW
CHUNK_ROWS = 4
CHUNK_TOK = CHUNK_ROWS * SEQ
NCHUNKS = ROWS_PER_W // CHUNK_ROWS
W_TOK = ROWS_PER_W * SEQ
FOLD_R = 5000
CNT_R = 512


def _fold_body(e0, e1, e2, e3, w_ref, o_ref):
    i = pl.program_id(0)
    lane = lax.broadcasted_iota(jnp.int32, (FOLD_R, CP), 1)
    row = lax.broadcasted_iota(jnp.int32, (FOLD_R, CP), 0)

    def mm(e_ref, j):
        g = lax.dot_general(
            e_ref[...], w_ref[...], (((1,), (1,)), ((), ())),
            preferred_element_type=jnp.float32)
        if j == 0:
            ind = jnp.where((lane == CLS) & ((row + i * FOLD_R) != 0),
                            1.0, 0.0)
        else:
            ind = jnp.where(lane == CLS, 1.0, 0.0)
        return g + ind

    o_ref[...] = jnp.concatenate(
        [mm(e0, 0), mm(e1, 1), mm(e2, 2), mm(e3, 3)], axis=1)


def _fold(embed, wp):
    q = VOCAB // 4 // FOLD_R
    return pl.pallas_call(
        _fold_body,
        grid=(q,),
        in_specs=[
            pl.BlockSpec((FOLD_R, EMBED), lambda i, j=j: (j * q + i, 0))
            for j in range(4)
        ] + [pl.BlockSpec((CP, EMBED), lambda i: (0, 0))],
        out_specs=pl.BlockSpec((FOLD_R, 4 * CP), lambda i: (i, 0)),
        out_shape=jax.ShapeDtypeStruct((VOCAB // 4, 4 * CP), jnp.float32),
    )(embed, embed, embed, embed, wp)


def _sc_pool(x_flat, folded, bp):
    mesh = plsc.VectorSubcoreMesh(
        core_axis_name="c", subcore_axis_name="s",
        num_cores=NC, num_subcores=NS)

    @functools.partial(
        pl.kernel,
        out_type=jax.ShapeDtypeStruct((BATCH, CP), jnp.float32),
        mesh=mesh,
        compiler_params=pltpu.CompilerParams(
            needs_layout_passes=False, use_tc_tiling_on_sc=False),
        scratch_types=[
            pltpu.VMEM((W_TOK,), jnp.int32),
            pltpu.VMEM((CHUNK_TOK, CP), jnp.float32),
            pltpu.VMEM((CHUNK_TOK, CP), jnp.float32),
            pltpu.VMEM((ROWS_PER_W, CP), jnp.float32),
            pltpu.VMEM((CP,), jnp.float32),
            pltpu.VMEM((16,), jnp.float32),
            pltpu.SemaphoreType.DMA,
            pltpu.SemaphoreType.DMA,
            pltpu.SemaphoreType.DMA,
            pltpu.SemaphoreType.DMA,
        ],
    )
    def body(x_hbm, folded_hbm, b_hbm, out_hbm, idx_v, rows_v0,
             rows_v1, out_v, b_v, recip_v, sem0, sem0b, sem1, sem1b):
        wid = lax.axis_index("s") * NC + lax.axis_index("c")
        pltpu.sync_copy(b_hbm, b_v)
        pltpu.sync_copy(x_hbm.at[pl.ds(wid * W_TOK, W_TOK)], idx_v)
        b0 = b_v[pl.ds(0, 16)]
        b1 = b_v[pl.ds(16, 16)]
        zeros16i = jnp.zeros((16,), jnp.int32)

        def stage(c):
            cb = c * CHUNK_TOK

            def stage_body(i, carry):
                for u in range(2):
                    v = idx_v[pl.ds(cb + i * 32 + u * 16, 16)]
                    one = jnp.int32(1)
                    zero = jnp.int32(0)
                    q = (jnp.where(v >= VOCAB // 4, one, zero)
                         + jnp.where(v >= VOCAB // 2, one, zero)
                         + jnp.where(v >= 3 * (VOCAB // 4), one, zero))
                    p = (v - q * (VOCAB // 4)) * 4 + q
                    idx_v[pl.ds(cb + i * 32 + u * 16, 16)] = p
                return carry

            lax.fori_loop(0, CHUNK_TOK // 32, stage_body, 0)

        H = CHUNK_TOK // 2

        def _halves(c, rows_v, sems):
            return [
                pltpu.make_async_copy(
                    folded_hbm.at[idx_v.at[pl.ds(c * CHUNK_TOK + h * H, H)]],
                    rows_v.at[pl.ds(h * H, H)], sems[h])
                for h in range(2)
            ]

        def start_gather(c, rows_v, sems):
            for cp in _halves(c, rows_v, sems):
                cp.start()

        def process(c, rows_v, sems):
            for cp in _halves(c, rows_v, sems):
                cp.wait()

            def tok_body(t, carry):
                accs = []
                for r in range(CHUNK_ROWS):
                    accs.append(carry[2 * r] + rows_v[r * SEQ + t, 0:16])
                    accs.append(carry[2 * r + 1] + rows_v[r * SEQ + t, 16:32])
                return tuple(accs)

            init = tuple(
                jnp.zeros((16,), jnp.float32) for _ in range(2 * CHUNK_ROWS))
            res = lax.fori_loop(0, SEQ, tok_body, init)
            for r in range(CHUNK_ROWS):
                row = c * CHUNK_ROWS + r
                recip_v[...] = res[2 * r + 1]
                rsp = 1.0 / plsc.load_gather(recip_v, [zeros16i + (CLS - 16)])
                out_v[row, 0:16] = res[2 * r] * rsp + b0
                out_v[row, 16:32] = res[2 * r + 1] * rsp + b1

        stage(0)
        start_gather(0, rows_v0, (sem0, sem0b))
        stage(1)

        def super_body(g, carry):
            start_gather(2 * g + 1, rows_v1, (sem1, sem1b))

            @pl.when(g < NCHUNKS // 2 - 1)
            def _():
                stage(2 * g + 2)

            process(2 * g, rows_v0, (sem0, sem0b))

            @pl.when(g < NCHUNKS // 2 - 1)
            def _():
                start_gather(2 * g + 2, rows_v0, (sem0, sem0b))
                stage(2 * g + 3)

            process(2 * g + 1, rows_v1, (sem1, sem1b))
            return carry

        lax.fori_loop(0, NCHUNKS // 2, super_body, 0)
        pltpu.sync_copy(out_v, out_hbm.at[pl.ds(wid * ROWS_PER_W,
                                                ROWS_PER_W)])

    return body(x_flat, folded, bp)


def kernel(X, embed_weight, W, b):
    x_flat = X.astype(jnp.int32).reshape(BATCH * SEQ)
    wp = jnp.zeros((CP, EMBED), jnp.float32).at[:CLS].set(W)
    bp = jnp.zeros((CP,), jnp.float32).at[:CLS].set(b)
    folded = _fold(embed_weight, wp).reshape(VOCAB, CP)
    out = _sc_pool(x_flat, folded, bp)
    return out[:, :CLS]


# speedup vs baseline: 1.7455x; 1.0025x over previous
import functools

import jax
import jax.numpy as jnp
from jax import lax
from jax.experimental import pallas as pl
from jax.experimental.pallas import tpu as pltpu
from jax.experimental.pallas import tpu_sc as plsc

BATCH = 4096
SEQ = 200
VOCAB = 100000
EMBED = 128
CLS = 20
CP = 32
NC, NS = 2, 16
NW = NC * NS
ROWS_PER_W = BATCH // NW
CHUNK_ROWS = 4
CHUNK_TOK = CHUNK_ROWS * SEQ
SEQP = 256
CHUNK_TOKP = CHUNK_ROWS * SEQP
NCHUNKS = ROWS_PER_W // CHUNK_ROWS
W_TOKP = ROWS_PER_W * SEQP
FOLD_R = 5000
CNT_R = 512


def _fold_body(e0, e1, e2, e3, w_ref, o_ref):
    i = pl.program_id(0)
    lane = lax.broadcasted_iota(jnp.int32, (FOLD_R, CP), 1)
    row = lax.broadcasted_iota(jnp.int32, (FOLD_R, CP), 0)

    def mm(e_ref, j):
        g = lax.dot_general(
            e_ref[...], w_ref[...], (((1,), (1,)), ((), ())),
            preferred_element_type=jnp.float32)
        if j == 0:
            ind = jnp.where((lane == CLS) & ((row + i * FOLD_R) != 0),
                            1.0, 0.0)
        else:
            ind = jnp.where(lane == CLS, 1.0, 0.0)
        return g + ind

    o_ref[...] = jnp.concatenate(
        [mm(e0, 0), mm(e1, 1), mm(e2, 2), mm(e3, 3)], axis=1)


def _fold(embed, wp):
    q = VOCAB // 4 // FOLD_R
    return pl.pallas_call(
        _fold_body,
        grid=(q,),
        in_specs=[
            pl.BlockSpec((FOLD_R, EMBED), lambda i, j=j: (j * q + i, 0))
            for j in range(4)
        ] + [pl.BlockSpec((CP, EMBED), lambda i: (0, 0))],
        out_specs=pl.BlockSpec((FOLD_R, 4 * CP), lambda i: (i, 0)),
        out_shape=jax.ShapeDtypeStruct((VOCAB // 4, 4 * CP), jnp.float32),
    )(embed, embed, embed, embed, wp)


def _sc_pool(x_flat, folded, bp):
    mesh = plsc.VectorSubcoreMesh(
        core_axis_name="c", subcore_axis_name="s",
        num_cores=NC, num_subcores=NS)

    @functools.partial(
        pl.kernel,
        out_type=jax.ShapeDtypeStruct((BATCH, CP), jnp.float32),
        mesh=mesh,
        compiler_params=pltpu.CompilerParams(
            needs_layout_passes=False, use_tc_tiling_on_sc=False),
        scratch_types=[
            pltpu.VMEM((W_TOKP,), jnp.int32),
            pltpu.VMEM((CHUNK_TOK, CP), jnp.float32),
            pltpu.VMEM((CHUNK_TOK, CP), jnp.float32),
            pltpu.VMEM((ROWS_PER_W, CP), jnp.float32),
            pltpu.VMEM((CP,), jnp.float32),
            pltpu.VMEM((16,), jnp.float32),
            pltpu.SemaphoreType.DMA,
            pltpu.SemaphoreType.DMA,
            pltpu.SemaphoreType.DMA,
            pltpu.SemaphoreType.DMA,
        ],
    )
    def body(x_hbm, folded_hbm, b_hbm, out_hbm, idx_v, rows_v0,
             rows_v1, out_v, b_v, recip_v, sem0, sem0b, sem1, sem1b):
        wid = lax.axis_index("s") * NC + lax.axis_index("c")
        pltpu.sync_copy(b_hbm, b_v)
        pltpu.sync_copy(x_hbm.at[pl.ds(wid * W_TOKP, W_TOKP)], idx_v)
        b0 = b_v[pl.ds(0, 16)]
        b1 = b_v[pl.ds(16, 16)]
        zeros16i = jnp.zeros((16,), jnp.int32)

        def stage(c):
            cb = c * CHUNK_TOKP

            def stage_body(i, carry):
                for u in range(2):
                    v = idx_v[pl.ds(cb + i * 32 + u * 16, 16)]
                    one = jnp.int32(1)
                    zero = jnp.int32(0)
                    q = (jnp.where(v >= VOCAB // 4, one, zero)
                         + jnp.where(v >= VOCAB // 2, one, zero)
                         + jnp.where(v >= 3 * (VOCAB // 4), one, zero))
                    p = (v - q * (VOCAB // 4)) * 4 + q
                    idx_v[pl.ds(cb + i * 32 + u * 16, 16)] = p
                return carry

            lax.fori_loop(0, CHUNK_TOKP // 32, stage_body, 0)

        def _segs(c, rows_v, sems):
            return [
                pltpu.make_async_copy(
                    folded_hbm.at[idx_v.at[pl.ds(c * CHUNK_TOKP + r * SEQP,
                                                 SEQ)]],
                    rows_v.at[pl.ds(r * SEQ, SEQ)], sems[r % 2])
                for r in range(CHUNK_ROWS)
            ]

        def start_gather(c, rows_v, sems):
            for cp in _segs(c, rows_v, sems):
                cp.start()

        def process(c, rows_v, sems):
            for cp in _segs(c, rows_v, sems):
                cp.wait()

            def tok_body(t, carry):
                accs = []
                for r in range(CHUNK_ROWS):
                    accs.append(carry[2 * r] + rows_v[r * SEQ + t, 0:16])
                    accs.append(carry[2 * r + 1] + rows_v[r * SEQ + t, 16:32])
                return tuple(accs)

            init = tuple(
                jnp.zeros((16,), jnp.float32) for _ in range(2 * CHUNK_ROWS))
            res = lax.fori_loop(0, SEQ, tok_body, init)
            for r in range(CHUNK_ROWS):
                row = c * CHUNK_ROWS + r
                recip_v[...] = res[2 * r + 1]
                rsp = 1.0 / plsc.load_gather(recip_v, [zeros16i + (CLS - 16)])
                out_v[row, 0:16] = res[2 * r] * rsp + b0
                out_v[row, 16:32] = res[2 * r + 1] * rsp + b1

        stage(0)
        start_gather(0, rows_v0, (sem0, sem0b))
        stage(1)

        def super_body(g, carry):
            start_gather(2 * g + 1, rows_v1, (sem1, sem1b))

            @pl.when(g < NCHUNKS // 2 - 1)
            def _():
                stage(2 * g + 2)

            process(2 * g, rows_v0, (sem0, sem0b))

            @pl.when(g < NCHUNKS // 2 - 1)
            def _():
                start_gather(2 * g + 2, rows_v0, (sem0, sem0b))
                stage(2 * g + 3)

            process(2 * g + 1, rows_v1, (sem1, sem1b))
            return carry

        lax.fori_loop(0, NCHUNKS // 2, super_body, 0)
        pltpu.sync_copy(out_v, out_hbm.at[pl.ds(wid * ROWS_PER_W,
                                                ROWS_PER_W)])

    return body(x_flat, folded, bp)


def kernel(X, embed_weight, W, b):
    xp = jnp.pad(X.astype(jnp.int32), ((0, 0), (0, SEQP - SEQ)))
    x_flat = xp.reshape(BATCH * SEQP)
    wp = jnp.zeros((CP, EMBED), jnp.float32).at[:CLS].set(W)
    bp = jnp.zeros((CP,), jnp.float32).at[:CLS].set(b)
    folded = _fold(embed_weight, wp).reshape(VOCAB, CP)
    out = _sc_pool(x_flat, folded, bp)
    return out[:, :CLS]
